# Initial kernel scaffold; baseline (speedup 1.0000x reference)
import functools

import jax
import jax.numpy as jnp
from jax import lax
from jax.experimental import pallas as pl
from jax.experimental.pallas import tpu as pltpu
from jax.experimental.pallas import tpu_sc as plsc

N_NODES = 10000
N_EDGES = 320000
D = 128

NC = 2
NS = 16
NW = NC * NS

LANES = 128
STREAMS = 79
E_PER_TILE = STREAMS * LANES
E_PAD = NW * E_PER_TILE
ACC_N = 10240
ZROWS = ACC_N // NS
OUT_ROWS = N_NODES // NS

_sc_mesh = plsc.VectorSubcoreMesh(core_axis_name="c", subcore_axis_name="s")


@functools.partial(
    pl.kernel,
    out_type=jax.ShapeDtypeStruct((NC, N_NODES, D), jnp.float32),
    mesh=_sc_mesh,
    scratch_types=[
        pltpu.VMEM((STREAMS, LANES), jnp.int32),
        pltpu.VMEM((STREAMS, LANES), jnp.int32),
        pltpu.VMEM((LANES, D), jnp.float32),
        pltpu.VMEM_SHARED((ACC_N, D), jnp.float32),
        pltpu.SemaphoreType.DMA,
    ],
)
def _sc_segment_sum(emb_hbm, src_hbm, dst_hbm, out_hbm,
                    src_v, dst_v, rows_v, acc_sh, sem):
    cid = lax.axis_index("c")
    sid = lax.axis_index("s")
    wid = cid * NS + sid

    def zero_row(i, _):
        for c in range(D // 16):
            rows_v[i, pl.ds(c * 16, 16)] = jnp.zeros((16,), jnp.float32)
        return 0
    lax.fori_loop(0, LANES, zero_row, 0)
    for z in range(ZROWS // LANES):
        pltpu.sync_copy(rows_v, acc_sh.at[pl.ds(sid * ZROWS + z * LANES, LANES)])
    plsc.subcore_barrier()

    pltpu.sync_copy(src_hbm.at[wid], src_v)
    pltpu.sync_copy(dst_hbm.at[wid], dst_v)

    def edge_body(j, _):
        pltpu.async_copy(emb_hbm.at[src_v.at[j]], rows_v, sem).wait()
        pltpu.sync_copy(rows_v, acc_sh.at[dst_v.at[j]], add=True)
        return 0
    lax.fori_loop(0, STREAMS, edge_body, 0)
    plsc.subcore_barrier()

    pltpu.sync_copy(acc_sh.at[pl.ds(sid * OUT_ROWS, OUT_ROWS)],
                    out_hbm.at[cid, pl.ds(sid * OUT_ROWS, OUT_ROWS)])


_BLK = 2000


def _tc_body(x_ref, n0_ref, n1_ref, w1_ref, w2_ref, b_ref, o_ref):
    h = jnp.dot(x_ref[...], w1_ref[...], preferred_element_type=jnp.float32)
    h += jnp.dot(n0_ref[...] + n1_ref[...], w2_ref[...],
                 preferred_element_type=jnp.float32)
    h += b_ref[...]
    h = jnp.maximum(h, 0.0)
    s = jnp.sum(h, axis=0, keepdims=True)

    @pl.when(pl.program_id(0) == 0)
    def _():
        o_ref[...] = jnp.zeros_like(o_ref)
    o_ref[...] += s


def _tc_reduce(x, nbr0, nbr1, W1T, W2T, bias):
    return pl.pallas_call(
        _tc_body,
        grid=(N_NODES // _BLK,),
        in_specs=[
            pl.BlockSpec((_BLK, D), lambda i: (i, 0)),
            pl.BlockSpec((_BLK, D), lambda i: (i, 0)),
            pl.BlockSpec((_BLK, D), lambda i: (i, 0)),
            pl.BlockSpec((D, D), lambda i: (0, 0)),
            pl.BlockSpec((D, D), lambda i: (0, 0)),
            pl.BlockSpec((1, D), lambda i: (0, 0)),
        ],
        out_specs=pl.BlockSpec((1, D), lambda i: (0, 0)),
        out_shape=jax.ShapeDtypeStruct((1, D), jnp.float32),
        compiler_params=pltpu.CompilerParams(
            dimension_semantics=("arbitrary",)),
    )(x, nbr0, nbr1, W1T, W2T, bias)


def kernel(x, edge_index, emb, W1, b1, W2, b2):
    src = edge_index[0]
    dst = edge_index[1]
    pad = E_PAD - N_EDGES
    src_p = jnp.concatenate([src, jnp.zeros((pad,), jnp.int32)])
    dst_p = jnp.concatenate([dst, jnp.full((pad,), N_NODES, jnp.int32)])
    src3 = src_p.reshape(NW, STREAMS, LANES)
    dst3 = dst_p.reshape(NW, STREAMS, LANES)

    partials = _sc_segment_sum(emb, src3, dst3)

    bias = (b1 + b2).reshape(1, D)
    out = _tc_reduce(x, partials[0], partials[1], W1.T, W2.T, bias)
    return out.reshape(D)


# speedup vs baseline: 5.0740x; 5.0740x over previous
import functools

import jax
import jax.numpy as jnp
from jax import lax
from jax.experimental import pallas as pl
from jax.experimental.pallas import tpu as pltpu
from jax.experimental.pallas import tpu_sc as plsc

N_NODES = 10000
N_EDGES = 320000
D = 128

NC = 2
NS = 16
NW = NC * NS

LANES = 128
STREAMS = 79
E_PER_TILE = STREAMS * LANES
E_PAD = NW * E_PER_TILE
ACC_N = 10240
ZROWS = ACC_N // NS

_sc_mesh = plsc.VectorSubcoreMesh(core_axis_name="c", subcore_axis_name="s")


@functools.partial(
    pl.kernel,
    out_type=jax.ShapeDtypeStruct((NC, ACC_N, D), jnp.float32),
    mesh=_sc_mesh,
    scratch_types=[
        pltpu.VMEM((STREAMS, LANES), jnp.int32),
        pltpu.VMEM((STREAMS, LANES), jnp.int32),
        pltpu.VMEM((LANES, D), jnp.float32),
        pltpu.VMEM_SHARED((ACC_N, D), jnp.float32),
        pltpu.SemaphoreType.DMA,
    ],
)
def _sc_segment_sum(emb_hbm, src_hbm, dst_hbm, out_hbm,
                    src_v, dst_v, rows_v, acc_sh, sem):
    cid = lax.axis_index("c")
    sid = lax.axis_index("s")
    wid = cid * NS + sid

    def zero_row(i, _):
        for c in range(D // 16):
            rows_v[i, pl.ds(c * 16, 16)] = jnp.zeros((16,), jnp.float32)
        return 0
    lax.fori_loop(0, LANES, zero_row, 0)
    for z in range(ZROWS // LANES):
        pltpu.sync_copy(rows_v, acc_sh.at[pl.ds(sid * ZROWS + z * LANES, LANES)])
    plsc.subcore_barrier()

    pltpu.sync_copy(src_hbm.at[wid], src_v)
    pltpu.sync_copy(dst_hbm.at[wid], dst_v)

    def edge_body(j, _):
        pltpu.async_copy(emb_hbm.at[src_v.at[j]], rows_v, sem).wait()
        pltpu.sync_copy(rows_v, acc_sh.at[dst_v.at[j]], add=True)
        return 0
    lax.fori_loop(0, STREAMS, edge_body, 0)
    plsc.subcore_barrier()

    pltpu.sync_copy(acc_sh.at[pl.ds(sid * ZROWS, ZROWS)],
                    out_hbm.at[cid, pl.ds(sid * ZROWS, ZROWS)])


_BLK = 2000


def _tc_body(x_ref, n0_ref, n1_ref, w1_ref, w2_ref, b_ref, o_ref):
    h = jnp.dot(x_ref[...], w1_ref[...], preferred_element_type=jnp.float32)
    h += jnp.dot(n0_ref[...] + n1_ref[...], w2_ref[...],
                 preferred_element_type=jnp.float32)
    h += b_ref[...]
    h = jnp.maximum(h, 0.0)
    s = jnp.sum(h, axis=0, keepdims=True)

    @pl.when(pl.program_id(0) == 0)
    def _():
        o_ref[...] = jnp.zeros_like(o_ref)
    o_ref[...] += s


def _tc_reduce(x, nbr0, nbr1, W1T, W2T, bias):
    return pl.pallas_call(
        _tc_body,
        grid=(N_NODES // _BLK,),
        in_specs=[
            pl.BlockSpec((_BLK, D), lambda i: (i, 0)),
            pl.BlockSpec((_BLK, D), lambda i: (i, 0)),
            pl.BlockSpec((_BLK, D), lambda i: (i, 0)),
            pl.BlockSpec((D, D), lambda i: (0, 0)),
            pl.BlockSpec((D, D), lambda i: (0, 0)),
            pl.BlockSpec((1, D), lambda i: (0, 0)),
        ],
        out_specs=pl.BlockSpec((1, D), lambda i: (0, 0)),
        out_shape=jax.ShapeDtypeStruct((1, D), jnp.float32),
        compiler_params=pltpu.CompilerParams(
            dimension_semantics=("arbitrary",)),
    )(x, nbr0, nbr1, W1T, W2T, bias)


def kernel(x, edge_index, emb, W1, b1, W2, b2):
    src = edge_index[0]
    dst = edge_index[1]
    pad = E_PAD - N_EDGES
    src_p = jnp.concatenate([src, jnp.zeros((pad,), jnp.int32)])
    dst_p = jnp.concatenate([dst, jnp.full((pad,), N_NODES, jnp.int32)])
    src3 = src_p.reshape(NW, STREAMS, LANES)
    dst3 = dst_p.reshape(NW, STREAMS, LANES)

    partials = _sc_segment_sum(emb, src3, dst3)

    bias = (b1 + b2).reshape(1, D)
    out = _tc_reduce(x, partials[0, :N_NODES], partials[1, :N_NODES],
                     W1.T, W2.T, bias)
    return out.reshape(D)
